# baseline (speedup 1.0000x reference)
import functools

import jax
import jax.numpy as jnp
from jax import lax
from jax.experimental import pallas as pl
from jax.experimental.pallas import tpu as pltpu

TOPK = 256


def _cnt_ge(bits, t):
    return jnp.sum((bits >= t).astype(jnp.int32), axis=1, keepdims=True)


def _indexer_body(q_ref, wq_ref, bq_ref, k_ref, out_ref):
    qrows = []
    for b in range(2):
        qrows.append(lax.dot_general(
            q_ref[b, :, 0, :, :].reshape(16, 1024), wq_ref[...],
            (((1,), (1,)), ((), ())),
            preferred_element_type=jnp.float32,
        ))
    q_down = jnp.concatenate(qrows, axis=0) + bq_ref[...]

    rows = []
    for b in range(2):
        qb = q_down[b * 16:(b + 1) * 16, :]
        rows.append(lax.dot_general(
            qb, k_ref[b], (((1,), (1,)), ((), ())),
            preferred_element_type=jnp.float32,
        ))
    scores = jnp.concatenate(rows, axis=0)

    fuzzy = jnp.maximum(scores, 0.0)
    bits = jnp.where(fuzzy > 0.0, lax.bitcast_convert_type(fuzzy, jnp.int32),
                     jnp.int32(0))

    def step(i, cand):
        sh = 29 - 2 * i
        d = ((_cnt_ge(bits, cand | (jnp.int32(1) << sh)) >= TOPK)
             .astype(jnp.int32)
             + (_cnt_ge(bits, cand | (jnp.int32(2) << sh)) >= TOPK)
             .astype(jnp.int32)
             + (_cnt_ge(bits, cand | (jnp.int32(3) << sh)) >= TOPK)
             .astype(jnp.int32))
        return cand | (d << sh)

    cand0 = jnp.zeros((bits.shape[0], 1), dtype=jnp.int32)
    kth = lax.fori_loop(0, 15, step, cand0)
    t = kth | jnp.int32(1)
    kth = jnp.where(_cnt_ge(bits, t) >= TOPK, t, kth)

    out_ref[...] = jnp.where(bits >= kth, fuzzy, 0.0)


@jax.jit
def _run(Q, Wq, bq, K):
    return pl.pallas_call(
        _indexer_body,
        grid=(1,),
        in_specs=[
            pl.BlockSpec((2, 16, 1, 8, 128), lambda c: (0, 0, 2047, 0, 0)),
            pl.BlockSpec((256, 1024), lambda c: (0, 0)),
            pl.BlockSpec((1, 256), lambda c: (0, 0)),
            pl.BlockSpec((2, 2048, 256), lambda c: (0, 0, 0)),
        ],
        out_specs=pl.BlockSpec((32, 2048), lambda c: (0, 0)),
        out_shape=jax.ShapeDtypeStruct((32, 2048), jnp.float32),
    )(Q, Wq, bq, K)


def kernel(Q, K_down, V_down, Wq, bq):
    K = K_down[:, 0, :, :]
    out = _run(Q.reshape(2, 16, 2048, 8, 128), Wq,
               bq.reshape(1, 256), K)
    return out.reshape(2, 16, 2048)


# speedup vs baseline: 28.4822x; 28.4822x over previous
import functools

import jax
import jax.numpy as jnp
from jax import lax
from jax.experimental import pallas as pl
from jax.experimental.pallas import tpu as pltpu

TOPK = 256


def _cnt_ge(bits, t):
    return jnp.sum((bits >= t).astype(jnp.int32), axis=1, keepdims=True)


def _indexer_body(q_ref, wq_ref, bq_ref, k_ref, out_ref):
    qrows = []
    for b in range(2):
        qrows.append(lax.dot_general(
            q_ref[b, :, 7, :], wq_ref[...], (((1,), (1,)), ((), ())),
            preferred_element_type=jnp.float32,
        ))
    q_down = jnp.concatenate(qrows, axis=0) + bq_ref[...]

    rows = []
    for b in range(2):
        qb = q_down[b * 16:(b + 1) * 16, :]
        rows.append(lax.dot_general(
            qb, k_ref[b], (((1,), (1,)), ((), ())),
            preferred_element_type=jnp.float32,
        ))
    scores = jnp.concatenate(rows, axis=0)

    fuzzy = jnp.maximum(scores, 0.0)
    bits = jnp.where(fuzzy > 0.0, lax.bitcast_convert_type(fuzzy, jnp.int32),
                     jnp.int32(0))

    def step(i, cand):
        sh = 29 - 2 * i
        d = ((_cnt_ge(bits, cand | (jnp.int32(1) << sh)) >= TOPK)
             .astype(jnp.int32)
             + (_cnt_ge(bits, cand | (jnp.int32(2) << sh)) >= TOPK)
             .astype(jnp.int32)
             + (_cnt_ge(bits, cand | (jnp.int32(3) << sh)) >= TOPK)
             .astype(jnp.int32))
        return cand | (d << sh)

    cand0 = jnp.zeros((bits.shape[0], 1), dtype=jnp.int32)
    kth = lax.fori_loop(0, 15, step, cand0)
    t = kth | jnp.int32(1)
    kth = jnp.where(_cnt_ge(bits, t) >= TOPK, t, kth)

    out_ref[...] = jnp.where(bits >= kth, fuzzy, 0.0)


@jax.jit
def _run(Q, Wq, bq, K):
    return pl.pallas_call(
        _indexer_body,
        grid=(1,),
        in_specs=[
            pl.BlockSpec((2, 16, 8, 1024), lambda c: (0, 0, 255, 0)),
            pl.BlockSpec((256, 1024), lambda c: (0, 0)),
            pl.BlockSpec((1, 256), lambda c: (0, 0)),
            pl.BlockSpec((2, 2048, 256), lambda c: (0, 0, 0)),
        ],
        out_specs=pl.BlockSpec((32, 2048), lambda c: (0, 0)),
        out_shape=jax.ShapeDtypeStruct((32, 2048), jnp.float32),
    )(Q, Wq, bq, K)


def kernel(Q, K_down, V_down, Wq, bq):
    K = K_down[:, 0, :, :]
    out = _run(Q, Wq, bq.reshape(1, 256), K)
    return out.reshape(2, 16, 2048)
